# Initial kernel scaffold; baseline (speedup 1.0000x reference)
import jax
import jax.numpy as jnp
from jax.experimental import pallas as pl


def kernel(x, u_table, W, b):
    raise NotImplementedError("write your pallas kernel here")


# speedup vs baseline: 1.1604x; 1.1604x over previous
import functools

import jax
import jax.numpy as jnp
from jax import lax
from jax.experimental import pallas as pl
from jax.experimental.pallas import tpu as pltpu
from jax.experimental.pallas import tpu_sc as plsc

LEN_USERS = 100000
EMBED_DIM = 128
BATCH = 16384

NUM_CORES = 2
NUM_SUBCORES = 16
LANES = 16
NUM_WORKERS = NUM_CORES * NUM_SUBCORES
BPW = BATCH // NUM_WORKERS
IDX_CHUNK = 128
NCHUNK = BPW // IDX_CHUNK


def _sc_kernel_body(uid_hbm, mov_hbm, table_hbm, wb_hbm, out_hbm,
                    idx_v, rows_v, mov_v, acc_v, wb_v, sem):
    wid = lax.axis_index("s") * NUM_CORES + lax.axis_index("c")
    base = wid * BPW

    for j in range(NCHUNK):
        pltpu.sync_copy(uid_hbm.at[pl.ds(base + j * IDX_CHUNK, IDX_CHUNK)],
                        idx_v.at[j])
    pltpu.sync_copy(mov_hbm.at[pl.ds(base, BPW)], mov_v)
    pltpu.sync_copy(wb_hbm, wb_v)

    copies = [
        pltpu.async_copy(table_hbm.at[idx_v.at[j]],
                         rows_v.at[pl.ds(j * IDX_CHUNK, IDX_CHUNK)], sem)
        for j in range(NCHUNK)
    ]
    for cp in copies:
        cp.wait()

    lane = lax.iota(jnp.int32, LANES)
    lane_masks = [lane == j for j in range(LANES)]
    last = jnp.full((LANES,), LANES - 1, jnp.int32)

    def group_body(g, _):
        row0 = g * LANES
        res = jnp.zeros((LANES,), jnp.float32)
        for j in range(LANES):
            acc = rows_v[row0 + j, pl.ds(0, LANES)]
            for c in range(1, EMBED_DIM // LANES):
                acc = acc + rows_v[row0 + j, pl.ds(c * LANES, LANES)]
            s_vec = plsc.cumsum(acc).at[last].get(mode="promise_in_bounds")
            res = jnp.where(lane_masks[j], s_vec, res)
        acc_v[pl.ds(row0, LANES)] = res
        return 0

    lax.fori_loop(0, BPW // LANES, group_body, 0)

    w_vec = wb_v[pl.ds(0, LANES)]
    b_vec = wb_v[pl.ds(LANES, LANES)]
    one = jnp.ones((LANES,), jnp.float32)
    for g in range(BPW // LANES):
        sl = pl.ds(g * LANES, LANES)
        z = acc_v[sl] * mov_v[sl] * w_vec + b_vec
        t = jnp.exp(-jnp.abs(z))
        acc_v[sl] = jnp.where(z >= 0, one / (one + t), t / (one + t))

    pltpu.sync_copy(acc_v, out_hbm.at[pl.ds(base, BPW)])


@jax.jit
def kernel(x, u_table, W, b):
    uid = x[:, 0]
    mov = x[:, 1].astype(jnp.float32)
    wb = jnp.concatenate([jnp.full((LANES,), W[0, 0], jnp.float32),
                          jnp.full((LANES,), b[0], jnp.float32)])

    mesh = plsc.VectorSubcoreMesh(core_axis_name="c", subcore_axis_name="s",
                                  num_cores=NUM_CORES,
                                  num_subcores=NUM_SUBCORES)
    run = functools.partial(
        pl.kernel,
        out_type=jax.ShapeDtypeStruct((BATCH,), jnp.float32),
        mesh=mesh,
        compiler_params=pltpu.CompilerParams(needs_layout_passes=False),
        scratch_types=[
            pltpu.VMEM((NCHUNK, IDX_CHUNK), jnp.int32),
            pltpu.VMEM((BPW, EMBED_DIM), jnp.float32),
            pltpu.VMEM((BPW,), jnp.float32),
            pltpu.VMEM((BPW,), jnp.float32),
            pltpu.VMEM((2 * LANES,), jnp.float32),
            pltpu.SemaphoreType.DMA,
        ],
    )(_sc_kernel_body)
    out = run(uid, mov, u_table, wb)
    return out.reshape(BATCH, 1)
